# baseline (speedup 1.0000x reference)
import jax
import jax.numpy as jnp
from jax.experimental import pallas as pl
from jax.experimental.pallas import tpu as pltpu

_BM = 2048
_NSLOT = 4
_NQ = 8


def _body(cw_ref, cb_ref, g_ref, b2_ref, x_hbm, o_hbm, xbuf, obuf, insem, outsem):
    B = x_hbm.shape[0]
    nsteps = B // _BM
    qrows = B // _NQ

    def in_copy(q):
        return pltpu.make_async_copy(
            x_hbm.at[pl.ds(q * qrows, qrows)],
            xbuf.at[pl.ds(q * qrows, qrows)],
            insem.at[q])

    def out_copy(i, slot):
        return pltpu.make_async_copy(
            obuf.at[slot], o_hbm.at[pl.ds(i * _BM, _BM)], outsem.at[slot])

    for q in range(_NQ):
        in_copy(q).start()

    steps_per_q = nsteps // _NQ

    def loop(i, carry):
        slot = jax.lax.rem(i, _NSLOT)
        @pl.when(jax.lax.rem(i, steps_per_q) == 0)
        def _():
            in_copy(jax.lax.div(i, steps_per_q)).wait()
        u = jnp.maximum(
            xbuf[pl.ds(i * _BM, _BM)] * cw_ref[...] + cb_ref[...], 0.0)
        acc = jnp.dot(u, g_ref[...], preferred_element_type=jnp.float32)
        @pl.when(i >= _NSLOT)
        def _():
            out_copy(i - _NSLOT, slot).wait()
        obuf[slot] = jnp.maximum(acc + b2_ref[...], 0.0)
        out_copy(i, slot).start()
        return carry

    jax.lax.fori_loop(0, nsteps, loop, 0)
    for j in range(max(0, nsteps - _NSLOT), nsteps):
        out_copy(j, j % _NSLOT).wait()


def kernel(x, conv_w, conv_b, W0, b0, W1, b1, W2, b2):
    B, I = x.shape
    N = W2.shape[0]
    K = W2.shape[1] // 2
    G = jnp.zeros((I, N), x.dtype).at[33::2, :].set(W2[:, :K].T)
    vmem = pl.BlockSpec(memory_space=pltpu.VMEM)
    hbm = pl.BlockSpec(memory_space=pl.ANY)
    out = pl.pallas_call(
        _body,
        in_specs=[vmem, vmem, vmem, vmem, hbm],
        out_specs=hbm,
        out_shape=jax.ShapeDtypeStruct((B, N), x.dtype),
        scratch_shapes=[
            pltpu.VMEM((B, I), x.dtype),
            pltpu.VMEM((_NSLOT, _BM, N), x.dtype),
            pltpu.SemaphoreType.DMA((_NQ,)),
            pltpu.SemaphoreType.DMA((_NSLOT,)),
        ],
    )(conv_w[None], conv_b[None], G, b2[None], x)
    return out


# speedup vs baseline: 1.0251x; 1.0251x over previous
import jax
import jax.numpy as jnp
from jax.experimental import pallas as pl
from jax.experimental.pallas import tpu as pltpu

_BM = 1024
_NSLOT = 6
_NQ = 8


def _body(cw_ref, cb_ref, g_ref, b2_ref, x_hbm, o_hbm, xbuf, obuf, insem, outsem):
    B = x_hbm.shape[0]
    nsteps = B // _BM
    qrows = B // _NQ

    def in_copy(q):
        return pltpu.make_async_copy(
            x_hbm.at[pl.ds(q * qrows, qrows)],
            xbuf.at[pl.ds(q * qrows, qrows)],
            insem.at[q])

    def out_copy(i, slot):
        return pltpu.make_async_copy(
            obuf.at[slot], o_hbm.at[pl.ds(i * _BM, _BM)], outsem.at[slot])

    for q in range(_NQ):
        in_copy(q).start()

    steps_per_q = nsteps // _NQ

    def loop(i, carry):
        slot = jax.lax.rem(i, _NSLOT)
        @pl.when(jax.lax.rem(i, steps_per_q) == 0)
        def _():
            in_copy(jax.lax.div(i, steps_per_q)).wait()
        u = jnp.maximum(
            xbuf[pl.ds(i * _BM, _BM)] * cw_ref[...] + cb_ref[...], 0.0)
        acc = jnp.dot(u, g_ref[...], preferred_element_type=jnp.float32)
        @pl.when(i >= _NSLOT)
        def _():
            out_copy(i - _NSLOT, slot).wait()
        obuf[slot] = jnp.maximum(acc + b2_ref[...], 0.0)
        out_copy(i, slot).start()
        return carry

    jax.lax.fori_loop(0, nsteps, loop, 0)
    for j in range(max(0, nsteps - _NSLOT), nsteps):
        out_copy(j, j % _NSLOT).wait()


def kernel(x, conv_w, conv_b, W0, b0, W1, b1, W2, b2):
    B, I = x.shape
    N = W2.shape[0]
    K = W2.shape[1] // 2
    G = jnp.zeros((I, N), x.dtype).at[33::2, :].set(W2[:, :K].T)
    vmem = pl.BlockSpec(memory_space=pltpu.VMEM)
    hbm = pl.BlockSpec(memory_space=pl.ANY)
    out = pl.pallas_call(
        _body,
        in_specs=[vmem, vmem, vmem, vmem, hbm],
        out_specs=hbm,
        out_shape=jax.ShapeDtypeStruct((B, N), x.dtype),
        scratch_shapes=[
            pltpu.VMEM((B, I), x.dtype),
            pltpu.VMEM((_NSLOT, _BM, N), x.dtype),
            pltpu.SemaphoreType.DMA((_NQ,)),
            pltpu.SemaphoreType.DMA((_NSLOT,)),
        ],
    )(conv_w[None], conv_b[None], G, b2[None], x)
    return out
